# baseline (speedup 1.0000x reference)
import jax
import jax.numpy as jnp
from jax.experimental import pallas as pl

import sc_kernel


def kernel(x):
    n, h, w, c = x.shape
    xt = x.transpose(0, 1, 3, 2).reshape(n * h, c, w)
    out = sc_kernel.sc_relu(xt)
    return out.reshape(n, h, c, w).transpose(0, 1, 3, 2)
